# baseline (speedup 1.0000x reference)
import functools

import jax
import jax.numpy as jnp
from jax import lax
from jax.experimental import pallas as pl
from jax.experimental.pallas import tpu as pltpu
from jax.experimental.pallas import tpu_sc as plsc

SRC = 100000
TGT = 50000
NNZ = 500000
D = 128
NCHUNK = 8
CW = 16
NSC = 2
NTILE = 16
K = 128
NNZP = 524288
NT = NNZP // NTILE
NB = NT // K
PB = 8
NOB2 = NB // (2 * PB)
TGTP = 50176
RPT = TGTP // NTILE
ZR = 112

_SC_MESH = plsc.VectorSubcoreMesh(core_axis_name="c", subcore_axis_name="s")


def _sc_body(xt, rows2, cols2, vals2, out, colsv, rowsv, valsv, bufs, zbuf,
             acc, gs0, gs1, gs2, gs3, ss0, ss1, ss2, ss3, is0, is1):
    core = lax.axis_index("c")
    sid = lax.axis_index("s")
    gsem = (gs0, gs1, gs2, gs3)
    ssem = (ss0, ss1, ss2, ss3)
    isem = (is0, is1)

    zv = jnp.zeros((16,), jnp.float32)

    def zero_zbuf(i, carry):
        for h in range(CW // 16):
            zbuf[i, h * 16:(h + 1) * 16] = zv
        return carry

    lax.fori_loop(0, ZR, zero_zbuf, 0)

    def chunk_body(cc, carry0):
        chunk = core * (NCHUNK // NSC) + cc
        table = xt.at[chunk]

        def idx_start(blk, slot):
            r0 = sid * NB + blk * PB
            pltpu.async_copy(cols2.at[pl.ds(r0, PB)], colsv.at[slot],
                             isem[slot])
            pltpu.async_copy(rows2.at[pl.ds(r0, PB)], rowsv.at[slot],
                             isem[slot])
            pltpu.async_copy(vals2.at[pl.ds(r0, PB)], valsv.at[slot],
                             isem[slot])

        def idx_wait(slot):
            r0 = sid * NB
            pltpu.make_async_copy(cols2.at[pl.ds(r0, PB)], colsv.at[slot],
                                  isem[slot]).wait()
            pltpu.make_async_copy(rows2.at[pl.ds(r0, PB)], rowsv.at[slot],
                                  isem[slot]).wait()
            pltpu.make_async_copy(vals2.at[pl.ds(r0, PB)], valsv.at[slot],
                                  isem[slot]).wait()

        def gather_start(slot, bb, p):
            pltpu.async_copy(table.at[colsv.at[slot, bb]], bufs.at[p],
                             gsem[p])

        def gather_wait(p):
            pltpu.make_async_copy(table.at[colsv.at[0, 0]], bufs.at[p],
                                  gsem[p]).wait()

        def scatter_start(slot, bb, p):
            pltpu.async_copy(bufs.at[p], acc.at[rowsv.at[slot, bb]], ssem[p],
                             add=True)

        def scatter_wait(p):
            pltpu.make_async_copy(bufs.at[p], acc.at[rowsv.at[0, 0]],
                                  ssem[p]).wait()

        idx_start(0, 0)
        idx_wait(0)
        gather_start(0, 0, 0)
        gather_start(0, 1, 1)

        def zero_acc(i, carry):
            pltpu.sync_copy(zbuf, acc.at[pl.ds(sid * RPT + i * ZR, ZR)])
            return carry

        lax.fori_loop(0, RPT // ZR, zero_acc, 0)
        plsc.subcore_barrier()

        def outer(ob, carry):
            for half in range(2):
                cur = half
                nxt = 1 - half
                for p in range(PB):
                    q = (p + 2) % 4
                    if half == 0 and p < 2:
                        @pl.when(ob >= 1)
                        def _():
                            scatter_wait(q)
                    else:
                        scatter_wait(q)
                    if p == 2:
                        if half == 0:
                            idx_start(2 * ob + 1, nxt)
                        else:
                            @pl.when(ob < NOB2 - 1)
                            def _():
                                idx_start(2 * ob + 2, nxt)
                    if p == PB - 2:
                        if half == 0:
                            idx_wait(nxt)
                        else:
                            @pl.when(ob < NOB2 - 1)
                            def _():
                                idx_wait(nxt)
                    if p < PB - 2:
                        gather_start(cur, p + 2, q)
                    elif half == 0:
                        gather_start(nxt, p - (PB - 2), q)
                    else:
                        @pl.when(ob < NOB2 - 1)
                        def _():
                            gather_start(nxt, p - (PB - 2), q)
                    gather_wait(p % 4)

                    def scale(g, c2):
                        vv = valsv[cur, p, pl.ds(g * 16, 16)]
                        for jj in range(16):
                            j = g * 16 + jj
                            v = vv[jj]
                            for h in range(CW // 16):
                                bufs[p % 4, j, h * 16:(h + 1) * 16] = (
                                    bufs[p % 4, j, h * 16:(h + 1) * 16] * v)
                        return c2

                    lax.fori_loop(0, K // 16, scale, 0)
                    scatter_start(cur, p, p % 4)
            return carry

        lax.fori_loop(0, NOB2, outer, 0)
        scatter_wait(2)
        scatter_wait(3)
        plsc.subcore_barrier()

        def write_out(i, carry):
            o = sid * RPT + i * ZR
            pltpu.sync_copy(acc.at[pl.ds(o, ZR)],
                            out.at[pl.ds(o, ZR), pl.ds(chunk * CW, CW)])
            return carry

        lax.fori_loop(0, RPT // ZR, write_out, 0)
        plsc.subcore_barrier()
        return carry0

    lax.fori_loop(0, NCHUNK // NSC, chunk_body, 0)


_sc_spmm = functools.partial(
    pl.kernel,
    out_type=jax.ShapeDtypeStruct((TGTP, D), jnp.float32),
    mesh=_SC_MESH,
    scratch_types=[
        pltpu.VMEM((2, PB, K), jnp.int32),
        pltpu.VMEM((2, PB, K), jnp.int32),
        pltpu.VMEM((2, PB, K), jnp.float32),
        pltpu.VMEM((4, K, CW), jnp.float32),
        pltpu.VMEM((ZR, CW), jnp.float32),
        pltpu.VMEM_SHARED((TGTP, CW), jnp.float32),
        pltpu.SemaphoreType.DMA,
        pltpu.SemaphoreType.DMA,
        pltpu.SemaphoreType.DMA,
        pltpu.SemaphoreType.DMA,
        pltpu.SemaphoreType.DMA,
        pltpu.SemaphoreType.DMA,
        pltpu.SemaphoreType.DMA,
        pltpu.SemaphoreType.DMA,
        pltpu.SemaphoreType.DMA,
        pltpu.SemaphoreType.DMA,
    ],
    compiler_params=pltpu.CompilerParams(use_tc_tiling_on_sc=False),
)(_sc_body)


RB = 2000


def _tc_body(mc_ref, w_ref, b_ref, g_ref, be_ref, o_ref):
    h = jnp.dot(mc_ref[...], w_ref[...], preferred_element_type=jnp.float32)
    h = h + b_ref[...]
    mean = jnp.mean(h, axis=-1, keepdims=True)
    cen = h - mean
    var = jnp.mean(cen * cen, axis=-1, keepdims=True)
    normed = cen * lax.rsqrt(var + 1e-5) * g_ref[...] + be_ref[...]
    o_ref[...] = normed * 0.5 * (1.0 + lax.erf(normed * 0.7071067811865476))


def _tc_dense(mc, w, b2, g2, be2):
    return pl.pallas_call(
        _tc_body,
        grid=(TGT // RB,),
        in_specs=[
            pl.BlockSpec((RB, D), lambda i: (i, 0)),
            pl.BlockSpec((D, D), lambda i: (0, 0)),
            pl.BlockSpec((1, D), lambda i: (0, 0)),
            pl.BlockSpec((1, D), lambda i: (0, 0)),
            pl.BlockSpec((1, D), lambda i: (0, 0)),
        ],
        out_specs=pl.BlockSpec((RB, D), lambda i: (i, 0)),
        out_shape=jax.ShapeDtypeStruct((TGT, D), jnp.float32),
    )(mc, w, b2, g2, be2)


def kernel(x, map_rows, map_cols, map_vals, W, b, gamma, beta):
    rows = map_rows.astype(jnp.int32)
    cols = map_cols.astype(jnp.int32)
    vals = map_vals.astype(jnp.float32)
    pad = NNZP - NNZ
    rows = jnp.concatenate([rows, jnp.zeros((pad,), jnp.int32)])
    cols = jnp.concatenate([cols, jnp.zeros((pad,), jnp.int32)])
    vals = jnp.concatenate([vals, jnp.zeros((pad,), jnp.float32)])
    rows2 = rows.reshape(NNZP // K, K)
    cols2 = cols.reshape(NNZP // K, K)
    vals2 = vals.reshape(NNZP // K, K)
    xt = x.reshape(SRC, NCHUNK, CW).transpose(1, 0, 2)
    mc = _sc_spmm(xt, rows2, cols2, vals2)
    return _tc_dense(mc, W, b.reshape(1, D), gamma.reshape(1, D),
                     beta.reshape(1, D))


# speedup vs baseline: 1.1521x; 1.1521x over previous
import functools

import numpy as np

import jax
import jax.numpy as jnp
from jax import lax
from jax.experimental import pallas as pl
from jax.experimental.pallas import tpu as pltpu
from jax.experimental.pallas import tpu_sc as plsc

SRC = 100000
TGT = 50000
NNZ = 500000
D = 128
NCHUNK = 4
CW = 32
NSC = 2
NTILE = 16
K = 128
NNZP = 524288
NT = NNZP // NTILE
NB = NT // K
PB = 8
NOB2 = NB // (2 * PB)
TGTP = 50176
RPT = TGTP // NTILE
ZR = 112

_SC_MESH = plsc.VectorSubcoreMesh(core_axis_name="c", subcore_axis_name="s")


def _sc_body(xt, rows2, cols2, vals2, out, colsv, rowsv, valsv, bufs,
             bufs16, zbuf, acc, gs0, gs1, gs2, gs3, ss0, ss1, ss2, ss3, is0, is1):
    core = lax.axis_index("c")
    sid = lax.axis_index("s")
    gsem = (gs0, gs1, gs2, gs3)
    ssem = (ss0, ss1, ss2, ss3)
    isem = (is0, is1)

    zv = jnp.zeros((32,), jnp.bfloat16)

    def zero_zbuf(i, carry):
        zbuf[i, 0:32] = zv
        return carry

    lax.fori_loop(0, ZR, zero_zbuf, 0)

    def chunk_body(cc, carry0):
        chunk = core * (NCHUNK // NSC) + cc
        table = xt.at[chunk]

        def idx_start(blk, slot):
            r0 = sid * NB + blk * PB
            pltpu.async_copy(cols2.at[pl.ds(r0, PB)], colsv.at[slot],
                             isem[slot])
            pltpu.async_copy(rows2.at[pl.ds(r0, PB)], rowsv.at[slot],
                             isem[slot])
            pltpu.async_copy(vals2.at[pl.ds(r0, PB)], valsv.at[slot],
                             isem[slot])

        def idx_wait(slot):
            r0 = sid * NB
            pltpu.make_async_copy(cols2.at[pl.ds(r0, PB)], colsv.at[slot],
                                  isem[slot]).wait()
            pltpu.make_async_copy(rows2.at[pl.ds(r0, PB)], rowsv.at[slot],
                                  isem[slot]).wait()
            pltpu.make_async_copy(vals2.at[pl.ds(r0, PB)], valsv.at[slot],
                                  isem[slot]).wait()

        def gather_start(slot, bb, p):
            pltpu.async_copy(table.at[colsv.at[slot, bb]], bufs.at[p],
                             gsem[p])

        def gather_wait(p):
            pltpu.make_async_copy(table.at[colsv.at[0, 0]], bufs.at[p],
                                  gsem[p]).wait()

        def scatter_start(slot, bb, p):
            pltpu.async_copy(bufs16.at[p], acc.at[rowsv.at[slot, bb]],
                             ssem[p], add=True)

        def scatter_wait(p):
            pltpu.make_async_copy(bufs16.at[p], acc.at[rowsv.at[0, 0]],
                                  ssem[p]).wait()

        idx_start(0, 0)
        idx_wait(0)
        gather_start(0, 0, 0)
        gather_start(0, 1, 1)

        def zero_acc(i, carry):
            pltpu.sync_copy(zbuf, acc.at[pl.ds(sid * RPT + i * ZR, ZR)])
            return carry

        lax.fori_loop(0, RPT // ZR, zero_acc, 0)
        plsc.subcore_barrier()

        def outer(ob, carry):
            for half in range(2):
                cur = half
                nxt = 1 - half
                for p in range(PB):
                    q = (p + 2) % 4
                    if half == 0 and p < 2:
                        @pl.when(ob >= 1)
                        def _():
                            scatter_wait(q)
                    else:
                        scatter_wait(q)
                    if p == 2:
                        if half == 0:
                            idx_start(2 * ob + 1, nxt)
                        else:
                            @pl.when(ob < NOB2 - 1)
                            def _():
                                idx_start(2 * ob + 2, nxt)
                    if p == PB - 2:
                        if half == 0:
                            idx_wait(nxt)
                        else:
                            @pl.when(ob < NOB2 - 1)
                            def _():
                                idx_wait(nxt)
                    if p < PB - 2:
                        gather_start(cur, p + 2, q)
                    elif half == 0:
                        gather_start(nxt, p - (PB - 2), q)
                    else:
                        @pl.when(ob < NOB2 - 1)
                        def _():
                            gather_start(nxt, p - (PB - 2), q)
                    gather_wait(p % 4)

                    def scale(g, c2):
                        vv = valsv[cur, p, pl.ds(g * 16, 16)]
                        for jj in range(16):
                            j = g * 16 + jj
                            v = vv[jj]
                            a = bufs[p % 4, j, 0:16] * v
                            b = bufs[p % 4, j, 16:32] * v
                            bufs16[p % 4, j, 0:32] = plsc.pack(
                                a, b, format=plsc.PackFormat.INTERLEAVED)
                        return c2

                    lax.fori_loop(0, K // 16, scale, 0)
                    scatter_start(cur, p, p % 4)
            return carry

        lax.fori_loop(0, NOB2, outer, 0)
        scatter_wait(2)
        scatter_wait(3)
        plsc.subcore_barrier()

        def write_out(i, carry):
            o = sid * RPT + i * ZR
            pltpu.sync_copy(acc.at[pl.ds(o, ZR)],
                            out.at[pl.ds(o, ZR), pl.ds(chunk * CW, CW)])
            return carry

        lax.fori_loop(0, RPT // ZR, write_out, 0)
        plsc.subcore_barrier()
        return carry0

    lax.fori_loop(0, NCHUNK // NSC, chunk_body, 0)


_sc_spmm = functools.partial(
    pl.kernel,
    out_type=jax.ShapeDtypeStruct((TGTP, D), jnp.bfloat16),
    mesh=_SC_MESH,
    scratch_types=[
        pltpu.VMEM((2, PB, K), jnp.int32),
        pltpu.VMEM((2, PB, K), jnp.int32),
        pltpu.VMEM((2, PB, K), jnp.float32),
        pltpu.VMEM((4, K, CW), jnp.float32),
        pltpu.VMEM((4, K, CW), jnp.bfloat16),
        pltpu.VMEM((ZR, CW), jnp.bfloat16),
        pltpu.VMEM_SHARED((TGTP, CW), jnp.bfloat16),
        pltpu.SemaphoreType.DMA,
        pltpu.SemaphoreType.DMA,
        pltpu.SemaphoreType.DMA,
        pltpu.SemaphoreType.DMA,
        pltpu.SemaphoreType.DMA,
        pltpu.SemaphoreType.DMA,
        pltpu.SemaphoreType.DMA,
        pltpu.SemaphoreType.DMA,
        pltpu.SemaphoreType.DMA,
        pltpu.SemaphoreType.DMA,
    ],
    compiler_params=pltpu.CompilerParams(use_tc_tiling_on_sc=False,
                                         needs_layout_passes=False),
)(_sc_body)


RB = 2000


def _tc_body(mc_ref, w_ref, b_ref, g_ref, be_ref, o_ref):
    m = mc_ref[...].astype(jnp.float32)
    h = jnp.dot(m, w_ref[...], preferred_element_type=jnp.float32)
    h = h + b_ref[...]
    mean = jnp.mean(h, axis=-1, keepdims=True)
    cen = h - mean
    var = jnp.mean(cen * cen, axis=-1, keepdims=True)
    normed = cen * lax.rsqrt(var + 1e-5) * g_ref[...] + be_ref[...]
    o_ref[...] = normed * 0.5 * (1.0 + lax.erf(normed * 0.7071067811865476))


def _tc_dense(mc, w, b2, g2, be2):
    return pl.pallas_call(
        _tc_body,
        grid=(TGT // RB,),
        in_specs=[
            pl.BlockSpec((RB, D), lambda i: (i, 0)),
            pl.BlockSpec((D, D), lambda i: (0, 0)),
            pl.BlockSpec((1, D), lambda i: (0, 0)),
            pl.BlockSpec((1, D), lambda i: (0, 0)),
            pl.BlockSpec((1, D), lambda i: (0, 0)),
        ],
        out_specs=pl.BlockSpec((RB, D), lambda i: (i, 0)),
        out_shape=jax.ShapeDtypeStruct((TGT, D), jnp.float32),
    )(mc, w, b2, g2, be2)


def kernel(x, map_rows, map_cols, map_vals, W, b, gamma, beta):
    rows = map_rows.astype(jnp.int32)
    cols = map_cols.astype(jnp.int32)
    vals = map_vals.astype(jnp.float32)
    pad = NNZP - NNZ
    rows = jnp.concatenate([rows, jnp.zeros((pad,), jnp.int32)])
    cols = jnp.concatenate([cols, jnp.zeros((pad,), jnp.int32)])
    vals = jnp.concatenate([vals, jnp.zeros((pad,), jnp.float32)])
    rows2 = rows.reshape(NNZP // K, K)
    cols2 = cols.reshape(NNZP // K, K)
    vals2 = vals.reshape(NNZP // K, K)
    xt = x.reshape(SRC, NCHUNK, CW).transpose(1, 0, 2)
    mc = _sc_spmm(xt, rows2, cols2, vals2)
    perm = np.array([k * CW + (q // 2) + 16 * (q % 2)
                     for k in range(NCHUNK) for q in range(CW)])
    return _tc_dense(mc, W[perm, :], b.reshape(1, D), gamma.reshape(1, D),
                     beta.reshape(1, D))
